# baseline (speedup 1.0000x reference)
import functools

import jax
import jax.numpy as jnp
from jax import lax
from jax.experimental import pallas as pl
from jax.experimental.pallas import tpu as pltpu
from jax.experimental.pallas import tpu_sc as plsc

_LANES = 16
_IDXW = 128
_E = 320000
_N = 10000
_IDX_ROWS = _E // _IDXW
N_PAD = 16 * 626
_P1_BASE = _IDX_ROWS // 16
_P1_REM = _IDX_ROWS - 16 * _P1_BASE
_P2_BASE = _IDX_ROWS // 32
_P2_REM = _IDX_ROWS - 32 * _P2_BASE
_CH = 13
_CHE = _CH * _IDXW
_P1_NCH = _P1_BASE // _CH
_P2_NCH = _P2_BASE // _CH


_ROWS128 = _E * _LANES // 128


@functools.partial(
    pl.kernel,
    out_type=jax.ShapeDtypeStruct((_E, _LANES), jnp.float32),
    mesh=plsc.VectorSubcoreMesh(
        core_axis_name="c", subcore_axis_name="s", num_cores=2, num_subcores=16
    ),
    scratch_types=[
        pltpu.VMEM_SHARED((N_PAD, _LANES), jnp.float32),
        pltpu.VMEM((2, _CHE, _LANES), jnp.float32),
        pltpu.VMEM((2, _CHE, _LANES), jnp.float32),
        pltpu.VMEM((3, _CH, 2, _IDXW), jnp.int32),
        pltpu.SemaphoreType.DMA,
        pltpu.SemaphoreType.DMA,
        pltpu.SemaphoreType.DMA,
        pltpu.SemaphoreType.DMA,
        pltpu.SemaphoreType.DMA,
    ],
    compiler_params=pltpu.CompilerParams(
        use_tc_tiling_on_sc=False, needs_layout_passes=False
    ),
)
def _sc_dmpnn(
    efeat_hbm, eidx_hbm, out_hbm, neigh, wbuf, gbuf, idx_v,
    sem_i, sem_w, sem_s, sem_g, sem_o,
):
    c = lax.axis_index("c")
    s = lax.axis_index("s")

    zrows = N_PAD // 16

    def _zero(i, carry):
        wbuf[0, i] = jnp.zeros((_LANES,), jnp.float32)
        return carry

    lax.fori_loop(0, zrows, _zero, 0)
    pltpu.sync_copy(
        wbuf.at[0, pl.ds(0, zrows)], neigh.at[pl.ds(s * zrows, zrows)]
    )
    plsc.subcore_barrier()

    def _fire_idx(k, rbase, n):
        return pltpu.async_copy(
            eidx_hbm.at[pl.ds(rbase, n)], idx_v.at[k % 3, pl.ds(0, n)], sem_i
        )

    def _fire_w(k, rbase, n):
        return pltpu.async_copy(
            efeat_hbm.at[pl.ds(rbase * _IDXW, n * _IDXW)],
            wbuf.at[k % 2, pl.ds(0, n * _IDXW)],
            sem_w,
        )

    def _fire_scatters(k, n):
        hs = []
        for j in range(n):
            hs.append(
                pltpu.async_copy(
                    wbuf.at[k % 2, pl.ds(j * _IDXW, _IDXW)],
                    neigh.at[idx_v.at[k % 3, j, 1]],
                    sem_s,
                    add=True,
                )
            )
        return hs

    p1_base = s * _P1_BASE
    loads = (_fire_idx(0, p1_base, _CH), _fire_w(0, p1_base, _CH))
    scats = []
    for k in range(_P1_NCH):
        for h in loads:
            h.wait()
        new_scats = _fire_scatters(k, _CH)
        for h in scats:
            h.wait()
        scats = new_scats
        if k + 1 < _P1_NCH:
            rb = p1_base + (k + 1) * _CH
            loads = (_fire_idx(k + 1, rb, _CH), _fire_w(k + 1, rb, _CH))
    for h in scats:
        h.wait()

    @pl.when(s < _P1_REM)
    def _p1_rem():
        row = 16 * _P1_BASE + s
        pltpu.sync_copy(eidx_hbm.at[pl.ds(row, 1)], idx_v.at[0, pl.ds(0, 1)])
        pltpu.sync_copy(
            efeat_hbm.at[pl.ds(row * _IDXW, _IDXW)],
            wbuf.at[0, pl.ds(0, _IDXW)],
        )
        pltpu.sync_copy(
            wbuf.at[0, pl.ds(0, _IDXW)], neigh.at[idx_v.at[0, 0, 1]], add=True
        )

    plsc.subcore_barrier()

    wid = c * 16 + s
    p2_base = wid * _P2_BASE

    def _fire_gathers(k, n):
        hs = []
        for j in range(n):
            hs.append(
                pltpu.async_copy(
                    neigh.at[idx_v.at[k % 3, j, 0]],
                    gbuf.at[k % 2, pl.ds(j * _IDXW, _IDXW)],
                    sem_g,
                )
            )
        return hs

    def _fire_store(k, rbase, n):
        return pltpu.async_copy(
            gbuf.at[k % 2, pl.ds(0, n * _IDXW)],
            out_hbm.at[pl.ds(rbase * _IDXW, n * _IDXW)],
            sem_o,
        )

    def _sub_chunk(q, n):
        wq = wbuf.at[q]
        gq = gbuf.at[q]

        def _body(t, carry):
            e = 8 * t
            for u in range(0, 8, 2):
                w_e = wq[e + u]
                w_o = wq[e + u + 1]
                gq[e + u] = gq[e + u] - w_o
                gq[e + u + 1] = gq[e + u + 1] - w_e
            return carry

        lax.fori_loop(0, n * (_IDXW // 8), _body, 0)

    idx_h = _fire_idx(0, p2_base, _CH)
    w_h = _fire_w(0, p2_base, _CH)
    st_h = [None, None]
    for k in range(_P2_NCH):
        idx_h.wait()
        if st_h[k % 2] is not None:
            st_h[k % 2].wait()
        g_h = _fire_gathers(k, _CH)
        if k + 1 < _P2_NCH:
            nrb = p2_base + (k + 1) * _CH
            next_idx = _fire_idx(k + 1, nrb, _CH)
            next_w = _fire_w(k + 1, nrb, _CH)
        for h in g_h:
            h.wait()
        w_h.wait()
        _sub_chunk(k % 2, _CH)
        st_h[k % 2] = _fire_store(k, p2_base + k * _CH, _CH)
        if k + 1 < _P2_NCH:
            idx_h = next_idx
            w_h = next_w
    for h in st_h:
        if h is not None:
            h.wait()

    @pl.when(wid < _P2_REM)
    def _p2_rem():
        row = 32 * _P2_BASE + wid
        pltpu.sync_copy(eidx_hbm.at[pl.ds(row, 1)], idx_v.at[0, pl.ds(0, 1)])
        pltpu.sync_copy(
            neigh.at[idx_v.at[0, 0, 0]], gbuf.at[0, pl.ds(0, _IDXW)]
        )
        pltpu.sync_copy(
            efeat_hbm.at[pl.ds(row * _IDXW, _IDXW)],
            wbuf.at[0, pl.ds(0, _IDXW)],
        )
        _sub_chunk(0, 1)
        pltpu.sync_copy(
            gbuf.at[0, pl.ds(0, _IDXW)],
            out_hbm.at[pl.ds(row * _IDXW, _IDXW)],
        )


_PBLK = 3200


def _pack_blk(src, dst):
    y = src[...].T.reshape(_PBLK // 8, 8, _LANES)
    dst[...] = jnp.concatenate([y[:, j, :] for j in range(8)], axis=1)


def _unpack_blk(src, dst):
    x = src[...]
    y = jnp.stack([x[:, 16 * j:16 * j + 16] for j in range(8)], axis=1)
    dst[...] = y.reshape(_PBLK, _LANES).T


_tc_pack = pl.pallas_call(
    _pack_blk,
    out_shape=jax.ShapeDtypeStruct((_ROWS128, 128), jnp.float32),
    grid=(_E // _PBLK,),
    in_specs=[pl.BlockSpec((_LANES, _PBLK), lambda i: (0, i))],
    out_specs=pl.BlockSpec((_PBLK // 8, 128), lambda i: (i, 0)),
)

_tc_unpack = pl.pallas_call(
    _unpack_blk,
    out_shape=jax.ShapeDtypeStruct((_LANES, _E), jnp.float32),
    grid=(_E // _PBLK,),
    in_specs=[pl.BlockSpec((_PBLK // 8, 128), lambda i: (i, 0))],
    out_specs=pl.BlockSpec((_LANES, _PBLK), lambda i: (0, i)),
)


def kernel(nfeat, efeat, edge_index):
    eidx = edge_index.reshape(2, _IDX_ROWS, _IDXW).transpose(1, 0, 2)
    ef2 = _tc_pack(efeat.T).reshape(_E, _LANES)
    out = _sc_dmpnn(ef2, eidx)
    return _tc_unpack(out.reshape(_ROWS128, 128)).T


# speedup vs baseline: 1.5572x; 1.5572x over previous
import functools

import jax
import jax.numpy as jnp
from jax import lax
from jax.experimental import pallas as pl
from jax.experimental.pallas import tpu as pltpu
from jax.experimental.pallas import tpu_sc as plsc

_LANES = 16
_IDXW = 128
_E = 320000
_N = 10000
_IDX_ROWS = _E // _IDXW
N_PAD = 16 * 626
_P1_BASE = _IDX_ROWS // 16
_P1_REM = _IDX_ROWS - 16 * _P1_BASE
_P2_BASE = _IDX_ROWS // 32
_P2_REM = _IDX_ROWS - 32 * _P2_BASE
_CH = 13
_CHE = _CH * _IDXW
_P1_NCH = _P1_BASE // _CH
_P2_NCH = _P2_BASE // _CH


@functools.partial(
    pl.kernel,
    out_type=jax.ShapeDtypeStruct((_E, _LANES), jnp.float32),
    mesh=plsc.VectorSubcoreMesh(
        core_axis_name="c", subcore_axis_name="s", num_cores=2, num_subcores=16
    ),
    scratch_types=[
        pltpu.VMEM_SHARED((N_PAD, _LANES), jnp.float32),
        pltpu.VMEM((2, _CHE, _LANES), jnp.float32),
        pltpu.VMEM((2, _CHE, _LANES), jnp.float32),
        pltpu.VMEM((3, _CH, 2, _IDXW), jnp.int32),
        pltpu.SemaphoreType.DMA,
        pltpu.SemaphoreType.DMA,
        pltpu.SemaphoreType.DMA,
        pltpu.SemaphoreType.DMA,
        pltpu.SemaphoreType.DMA,
    ],
    compiler_params=pltpu.CompilerParams(
        use_tc_tiling_on_sc=False, needs_layout_passes=False
    ),
)
def _sc_dmpnn(
    efeat_hbm, eidx_hbm, out_hbm, neigh, wbuf, gbuf, idx_v,
    sem_i, sem_w, sem_s, sem_g, sem_o,
):
    c = lax.axis_index("c")
    s = lax.axis_index("s")

    zrows = N_PAD // 16

    def _zero(i, carry):
        wbuf[0, i] = jnp.zeros((_LANES,), jnp.float32)
        return carry

    lax.fori_loop(0, zrows, _zero, 0)
    pltpu.sync_copy(
        wbuf.at[0, pl.ds(0, zrows)], neigh.at[pl.ds(s * zrows, zrows)]
    )
    plsc.subcore_barrier()

    def _fire_idx(k, rbase, n):
        return pltpu.async_copy(
            eidx_hbm.at[pl.ds(rbase, n)], idx_v.at[k % 3, pl.ds(0, n)], sem_i
        )

    def _fire_w(k, rbase, n):
        return pltpu.async_copy(
            efeat_hbm.at[pl.ds(rbase * _IDXW, n * _IDXW)],
            wbuf.at[k % 2, pl.ds(0, n * _IDXW)],
            sem_w,
        )

    def _fire_scatters(k, n):
        hs = []
        for j in range(n):
            hs.append(
                pltpu.async_copy(
                    wbuf.at[k % 2, pl.ds(j * _IDXW, _IDXW)],
                    neigh.at[idx_v.at[k % 3, j, 1]],
                    sem_s,
                    add=True,
                )
            )
        return hs

    p1_base = s * _P1_BASE
    loads = (_fire_idx(0, p1_base, _CH), _fire_w(0, p1_base, _CH))
    scats = []
    for k in range(_P1_NCH):
        for h in loads:
            h.wait()
        new_scats = _fire_scatters(k, _CH)
        for h in scats:
            h.wait()
        scats = new_scats
        if k + 1 < _P1_NCH:
            rb = p1_base + (k + 1) * _CH
            loads = (_fire_idx(k + 1, rb, _CH), _fire_w(k + 1, rb, _CH))
    for h in scats:
        h.wait()

    @pl.when(s < _P1_REM)
    def _p1_rem():
        row = 16 * _P1_BASE + s
        pltpu.sync_copy(eidx_hbm.at[pl.ds(row, 1)], idx_v.at[0, pl.ds(0, 1)])
        pltpu.sync_copy(
            efeat_hbm.at[pl.ds(row * _IDXW, _IDXW)],
            wbuf.at[0, pl.ds(0, _IDXW)],
        )
        pltpu.sync_copy(
            wbuf.at[0, pl.ds(0, _IDXW)], neigh.at[idx_v.at[0, 0, 1]], add=True
        )

    plsc.subcore_barrier()

    wid = c * 16 + s
    p2_base = wid * _P2_BASE

    def _fire_gathers(k, n):
        hs = []
        for j in range(n):
            hs.append(
                pltpu.async_copy(
                    neigh.at[idx_v.at[k % 3, j, 0]],
                    gbuf.at[k % 2, pl.ds(j * _IDXW, _IDXW)],
                    sem_g,
                )
            )
        return hs

    def _fire_store(k, rbase, n):
        return pltpu.async_copy(
            gbuf.at[k % 2, pl.ds(0, n * _IDXW)],
            out_hbm.at[pl.ds(rbase * _IDXW, n * _IDXW)],
            sem_o,
        )

    def _sub_chunk(q, n):
        wq = wbuf.at[q]
        gq = gbuf.at[q]

        def _body(t, carry):
            e = 8 * t
            for u in range(0, 8, 2):
                w_e = wq[e + u]
                w_o = wq[e + u + 1]
                gq[e + u] = gq[e + u] - w_o
                gq[e + u + 1] = gq[e + u + 1] - w_e
            return carry

        lax.fori_loop(0, n * (_IDXW // 8), _body, 0)

    idx_h = _fire_idx(0, p2_base, _CH)
    w_h = _fire_w(0, p2_base, _CH)
    st_h = [None, None]
    for k in range(_P2_NCH):
        idx_h.wait()
        if st_h[k % 2] is not None:
            st_h[k % 2].wait()
        g_h = _fire_gathers(k, _CH)
        if k + 1 < _P2_NCH:
            nrb = p2_base + (k + 1) * _CH
            next_idx = _fire_idx(k + 1, nrb, _CH)
            next_w = _fire_w(k + 1, nrb, _CH)
        for h in g_h:
            h.wait()
        w_h.wait()
        _sub_chunk(k % 2, _CH)
        st_h[k % 2] = _fire_store(k, p2_base + k * _CH, _CH)
        if k + 1 < _P2_NCH:
            idx_h = next_idx
            w_h = next_w
    for h in st_h:
        if h is not None:
            h.wait()

    @pl.when(wid < _P2_REM)
    def _p2_rem():
        row = 32 * _P2_BASE + wid
        pltpu.sync_copy(eidx_hbm.at[pl.ds(row, 1)], idx_v.at[0, pl.ds(0, 1)])
        pltpu.sync_copy(
            neigh.at[idx_v.at[0, 0, 0]], gbuf.at[0, pl.ds(0, _IDXW)]
        )
        pltpu.sync_copy(
            efeat_hbm.at[pl.ds(row * _IDXW, _IDXW)],
            wbuf.at[0, pl.ds(0, _IDXW)],
        )
        _sub_chunk(0, 1)
        pltpu.sync_copy(
            gbuf.at[0, pl.ds(0, _IDXW)],
            out_hbm.at[pl.ds(row * _IDXW, _IDXW)],
        )


def kernel(nfeat, efeat, edge_index):
    eidx = edge_index.reshape(2, _IDX_ROWS, _IDXW).transpose(1, 0, 2)
    return _sc_dmpnn(efeat, eidx)
